# baseline (speedup 1.0000x reference)
import functools

import jax
import jax.numpy as jnp
from jax import lax
from jax.experimental import pallas as pl
from jax.experimental.pallas import tpu as pltpu
from jax.experimental.pallas import tpu_sc as plsc

EPS = 1e-5
K = 16
_NC = 2
_NS = 16
_NW = _NC * _NS


def _pick(n, prefs):
    for p in prefs:
        if n % p == 0:
            return p
    return n


@functools.lru_cache(None)
def _sc_gather_kernel(V, D, B):
    b_per_w = B // _NW
    ch = None
    for c in (320, 256, 200, 160, 80, 40, 8):
        if b_per_w % c == 0 and c * D * 4 <= 360_000:
            ch = c
            break
    if ch is None:
        ch = b_per_w
    n_chunks = b_per_w // ch
    mesh = plsc.VectorSubcoreMesh(core_axis_name="c", subcore_axis_name="s")

    @functools.partial(
        pl.kernel,
        mesh=mesh,
        out_type=jax.ShapeDtypeStruct((B, D), jnp.float32),
        scratch_types=[
            pltpu.VMEM((ch,), jnp.int32),
            pltpu.VMEM((ch, D), jnp.float32),
            pltpu.SemaphoreType.DMA,
        ],
    )
    def k(table_hbm, idx_hbm, out_hbm, idx_v, rows_v, sem):
        wid = lax.axis_index("s") * _NC + lax.axis_index("c")
        base = wid * b_per_w

        def body(c, carry):
            off = pl.multiple_of(base + c * ch, 8)
            pltpu.sync_copy(idx_hbm.at[pl.ds(off, ch)], idx_v)
            pltpu.async_copy(table_hbm.at[idx_v], rows_v, sem).wait()
            pltpu.sync_copy(rows_v, out_hbm.at[pl.ds(off, ch)])
            return carry

        lax.fori_loop(0, n_chunks, body, 0)

    return k


def _gather_rows(table, idx):
    V, D = table.shape
    (B,) = idx.shape
    return _sc_gather_kernel(V, D, B)(table, idx)


def _proj(x, WaT, ba8, WvT, bv8):
    N, F = x.shape
    H = WaT.shape[1]
    BN = _pick(N, (200, 8))

    def body(x_ref, wa_ref, ba_ref, wv_ref, bv_ref, o_ref):
        i = pl.program_id(0)
        xb = x_ref[...]
        a = jnp.dot(xb, wa_ref[...], preferred_element_type=jnp.float32) + ba_ref[0:1, :]
        v = jnp.dot(xb, wv_ref[...], preferred_element_type=jnp.float32) + bv_ref[0:1, :]
        rows = i * BN + lax.broadcasted_iota(jnp.int32, (BN, 1), 0)
        o_ref[...] = jnp.where(rows % 5 == 0, a, v)

    return pl.pallas_call(
        body,
        grid=(N // BN,),
        in_specs=[
            pl.BlockSpec((BN, F), lambda i: (i, 0)),
            pl.BlockSpec((F, H), lambda i: (0, 0)),
            pl.BlockSpec((8, H), lambda i: (0, 0)),
            pl.BlockSpec((F, H), lambda i: (0, 0)),
            pl.BlockSpec((8, H), lambda i: (0, 0)),
        ],
        out_specs=pl.BlockSpec((BN, H), lambda i: (i, 0)),
        out_shape=jax.ShapeDtypeStruct((N, H), jnp.float32),
    )(x, WaT, ba8, WvT, bv8)


def _topk_idx(y, yT_pad, brow, bcol8, N, Npad):
    H = y.shape[1]
    BR = 8
    CH = 2560 if Npad % 2560 == 0 else Npad
    NCHUNK = Npad // CH
    NRB = N // BR

    def body(y_ref, yt_ref, br_ref, bc_ref, o_ref, rv_ref, ri_ref, sqc_ref):
        j = pl.program_id(0)
        i = pl.program_id(1)
        yb = y_ref[...]
        yt = yt_ref[...]

        @pl.when(i == 0)
        def _():
            sqc_ref[j, 0:1, :] = jnp.sum(yt * yt, axis=0, keepdims=True)

        @pl.when(j == 0)
        def _():
            rv_ref[pl.ds(i * BR, BR), :] = jnp.full((BR, K), jnp.inf, jnp.float32)
            ri_ref[pl.ds(i * BR, BR), :] = jnp.zeros((BR, K), jnp.int32)

        sqr = jnp.sum(yb * yb, axis=1, keepdims=True)
        d = sqr + sqc_ref[j, 0:1, :] - 2.0 * jnp.dot(
            yb, yt, preferred_element_type=jnp.float32)
        cols = j * CH + lax.broadcasted_iota(jnp.int32, (BR, CH), 1)
        d = jnp.where(br_ref[...] != bc_ref[0:1, :], 1e30, d)
        d = jnp.where(cols >= N, jnp.inf, d)

        rv = rv_ref[pl.ds(i * BR, BR), :]
        ri = ri_ref[pl.ds(i * BR, BR), :]
        dc = jnp.concatenate([d, rv], axis=1)
        ic = jnp.concatenate([cols, ri], axis=1)
        lane = lax.broadcasted_iota(jnp.int32, (BR, K), 1)

        def ext(k, carry):
            dcc, nv, ni = carry
            m = jnp.min(dcc, axis=1, keepdims=True)
            pick = jnp.min(
                jnp.where(dcc == m, ic, jnp.int32(2**30)), axis=1, keepdims=True)
            nv = jnp.where(lane == k, m, nv)
            ni = jnp.where(lane == k, pick, ni)
            dcc = jnp.where(ic == pick, jnp.inf, dcc)
            return dcc, nv, ni

        _, nv, ni = lax.fori_loop(
            0, K, ext,
            (dc, jnp.zeros((BR, K), jnp.float32), jnp.zeros((BR, K), jnp.int32)))
        rv_ref[pl.ds(i * BR, BR), :] = nv
        ri_ref[pl.ds(i * BR, BR), :] = ni

        @pl.when(j == NCHUNK - 1)
        def _():
            o_ref[...] = ni

    return pl.pallas_call(
        body,
        grid=(NCHUNK, NRB),
        in_specs=[
            pl.BlockSpec((BR, H), lambda j, i: (i, 0)),
            pl.BlockSpec((H, CH), lambda j, i: (0, j)),
            pl.BlockSpec((BR, 1), lambda j, i: (i, 0)),
            pl.BlockSpec((8, CH), lambda j, i: (0, j)),
        ],
        out_specs=pl.BlockSpec((BR, K), lambda j, i: (i, 0)),
        out_shape=jax.ShapeDtypeStruct((N, K), jnp.int32),
        scratch_shapes=[
            pltpu.VMEM((N, K), jnp.float32),
            pltpu.VMEM((N, K), jnp.int32),
            pltpu.VMEM((NCHUNK, 8, CH), jnp.float32),
        ],
        compiler_params=pltpu.CompilerParams(
            dimension_semantics=("arbitrary", "arbitrary")),
    )(y, yT_pad, brow, bcol8)


def _stats_dyn(y, xj, m8):
    N, H = y.shape
    BM = _pick(N, (200, 8))

    def body(y_ref, xj_ref, m_ref, o_ref):
        i = pl.program_id(0)
        yb0 = y_ref[...]
        ya = yb0 - m_ref[0:1, :]
        mb = m_ref[1:2, :]
        sb = jnp.zeros((1, H), jnp.float32)
        qb = jnp.zeros((1, H), jnp.float32)
        for k in range(K):
            b = xj_ref[:, k * H:(k + 1) * H] - yb0 - mb
            sb = sb + jnp.sum(b, axis=0, keepdims=True)
            qb = qb + jnp.sum(b * b, axis=0, keepdims=True)
        z = jnp.concatenate([
            K * jnp.sum(ya * ya, axis=0, keepdims=True),
            qb,
            K * jnp.sum(ya, axis=0, keepdims=True),
            sb,
            jnp.zeros((4, H), jnp.float32),
        ], axis=0)

        @pl.when(i == 0)
        def _():
            o_ref[...] = jnp.zeros((8, H), jnp.float32)

        o_ref[...] += z

    return pl.pallas_call(
        body,
        grid=(N // BM,),
        in_specs=[
            pl.BlockSpec((BM, H), lambda i: (i, 0)),
            pl.BlockSpec((BM, K * H), lambda i: (i, 0)),
            pl.BlockSpec((8, H), lambda i: (0, 0)),
        ],
        out_specs=pl.BlockSpec((8, H), lambda i: (0, 0)),
        out_shape=jax.ShapeDtypeStruct((8, H), jnp.float32),
        compiler_params=pltpu.CompilerParams(
            dimension_semantics=("arbitrary",)),
    )(y, xj, m8)


def _stats_static(zi, zj, m8):
    E, C = zi.shape
    BE = _pick(E, (2000, 8))

    def body(zi_ref, zj_ref, m_ref, o_ref):
        i = pl.program_id(0)
        a = zi_ref[...] - m_ref[0:1, :]
        b = zj_ref[...] - zi_ref[...] - m_ref[1:2, :]
        z = jnp.concatenate([
            jnp.sum(a * a, axis=0, keepdims=True),
            jnp.sum(b * b, axis=0, keepdims=True),
            jnp.sum(a, axis=0, keepdims=True),
            jnp.sum(b, axis=0, keepdims=True),
            jnp.zeros((4, C), jnp.float32),
        ], axis=0)

        @pl.when(i == 0)
        def _():
            o_ref[...] = jnp.zeros((8, C), jnp.float32)

        o_ref[...] += z

    return pl.pallas_call(
        body,
        grid=(E // BE,),
        in_specs=[
            pl.BlockSpec((BE, C), lambda i: (i, 0)),
            pl.BlockSpec((BE, C), lambda i: (i, 0)),
            pl.BlockSpec((8, C), lambda i: (0, 0)),
        ],
        out_specs=pl.BlockSpec((8, C), lambda i: (0, 0)),
        out_shape=jax.ShapeDtypeStruct((8, C), jnp.float32),
        compiler_params=pltpu.CompilerParams(
            dimension_semantics=("arbitrary",)),
    )(zi, zj, m8)


def _bn_apply_a(x, st_ref):
    return ((x - st_ref[0:1, :]) * lax.rsqrt(st_ref[1:2, :] + EPS)
            ) * st_ref[4:5, :] + st_ref[5:6, :]


def _bn_apply_b(x, st_ref):
    return ((x - st_ref[2:3, :]) * lax.rsqrt(st_ref[3:4, :] + EPS)
            ) * st_ref[6:7, :] + st_ref[7:8, :]


def _msg_dyn(y, xj, stats, WT):
    N, H = y.shape
    BM = _pick(N, (200, 8))

    def body(y_ref, xj_ref, st_ref, w_ref, o_ref):
        yb = y_ref[...]
        ra = jnp.maximum(_bn_apply_a(yb, st_ref), 0.0)
        out = None
        for k in range(K):
            b = xj_ref[:, k * H:(k + 1) * H] - yb
            rb = jnp.maximum(_bn_apply_b(b, st_ref), 0.0)
            mk = jnp.dot(jnp.concatenate([ra, rb], axis=1), w_ref[...],
                         preferred_element_type=jnp.float32)
            out = mk if out is None else jnp.maximum(out, mk)
        o_ref[...] = out

    return pl.pallas_call(
        body,
        grid=(N // BM,),
        in_specs=[
            pl.BlockSpec((BM, H), lambda i: (i, 0)),
            pl.BlockSpec((BM, K * H), lambda i: (i, 0)),
            pl.BlockSpec((8, H), lambda i: (0, 0)),
            pl.BlockSpec((2 * H, H), lambda i: (0, 0)),
        ],
        out_specs=pl.BlockSpec((BM, H), lambda i: (i, 0)),
        out_shape=jax.ShapeDtypeStruct((N, H), jnp.float32),
    )(y, xj, stats, WT)


def _msg_static(zi, zj, dst2d, stats, WT):
    E, C = zi.shape
    H = WT.shape[1]
    BE = _pick(E, (2000, 8))

    def body(zi_ref, zj_ref, d_ref, st_ref, w_ref, o_ref,
             cv_ref, cd_ref):
        i = pl.program_id(0)
        a = zi_ref[...]
        b = zj_ref[...] - a
        ra = jnp.maximum(_bn_apply_a(a, st_ref), 0.0)
        rb = jnp.maximum(_bn_apply_b(b, st_ref), 0.0)
        m = jnp.dot(jnp.concatenate([ra, rb], axis=1), w_ref[...],
                    preferred_element_type=jnp.float32)
        d = d_ref[...]

        @pl.when(i == 0)
        def _():
            cd_ref[0] = -1
            cv_ref[...] = jnp.full((8, H), -jnp.inf, jnp.float32)

        d0 = d_ref[0, 0]
        carry_hit = (d == d0) & (cd_ref[0] == d0)
        m = jnp.where(carry_hit, jnp.maximum(m, cv_ref[0:1, :]), m)
        rows = lax.broadcasted_iota(jnp.int32, (BE, 1), 0)
        s = 1
        while s < BE:
            pm = jnp.roll(m, s, axis=0)
            pd = jnp.roll(d, s, axis=0)
            ok = (rows >= s) & (pd == d)
            m = jnp.where(ok, jnp.maximum(m, pm), m)
            s *= 2
        o_ref[...] = m
        cv_ref[0:1, :] = m[BE - 1:BE, :]
        cd_ref[0] = d[BE - 1, 0]

    return pl.pallas_call(
        body,
        grid=(E // BE,),
        in_specs=[
            pl.BlockSpec((BE, C), lambda i: (i, 0)),
            pl.BlockSpec((BE, C), lambda i: (i, 0)),
            pl.BlockSpec((BE, 1), lambda i: (i, 0)),
            pl.BlockSpec((8, C), lambda i: (0, 0)),
            pl.BlockSpec((2 * C, H), lambda i: (0, 0)),
        ],
        out_specs=pl.BlockSpec((BE, H), lambda i: (i, 0)),
        out_shape=jax.ShapeDtypeStruct((E, H), jnp.float32),
        scratch_shapes=[
            pltpu.VMEM((8, H), jnp.float32),
            pltpu.SMEM((1,), jnp.int32),
        ],
        compiler_params=pltpu.CompilerParams(
            dimension_semantics=("arbitrary",)),
    )(zi, zj, dst2d, stats, WT)


def _fc(x4c, WfcT8, bfc8):
    N, C = x4c.shape
    BN = _pick(N, (200, 8))

    def body(x_ref, w_ref, b_ref, o_ref):
        o_ref[...] = jnp.dot(
            x_ref[...], w_ref[...], preferred_element_type=jnp.float32
        ) + b_ref[0:1, :]

    return pl.pallas_call(
        body,
        grid=(N // BN,),
        in_specs=[
            pl.BlockSpec((BN, C), lambda i: (i, 0)),
            pl.BlockSpec((C, 8), lambda i: (0, 0)),
            pl.BlockSpec((8, 8), lambda i: (0, 0)),
        ],
        out_specs=pl.BlockSpec((BN, 8), lambda i: (i, 0)),
        out_shape=jax.ShapeDtypeStruct((N, 8), jnp.float32),
    )(x4c, WfcT8, bfc8)


def _pad8(v):
    return jnp.concatenate(
        [v[None, :], jnp.zeros((7, v.shape[0]), jnp.float32)], axis=0)


def _bn_stats(sums_fn, data, cnt, g, b, C):
    s1 = sums_fn(*data, jnp.zeros((8, C), jnp.float32))
    mean_a = s1[2:3] / cnt
    mean_b = s1[3:4] / cnt
    m8 = jnp.concatenate([mean_a, mean_b, jnp.zeros((6, C), jnp.float32)], axis=0)
    s2 = sums_fn(*data, m8)
    params = jnp.stack([g[:C], b[:C], g[C:], b[C:]], axis=0)
    return jnp.concatenate(
        [mean_a, s2[0:1] / cnt, mean_b, s2[1:2] / cnt, params], axis=0)


def _dyn_conv(y, brow, bcol8, dg, db, dW, N, Npad):
    H = y.shape[1]
    idx = _topk_idx(y, jnp.pad(y.T, ((0, 0), (0, Npad - N))), brow, bcol8, N, Npad)
    xjw = _gather_rows(y, idx.reshape(-1))
    xi = jnp.broadcast_to(y[:, None, :], (N, K, H))
    xj3 = xjw.reshape(N, K, H)
    f = jnp.concatenate([xi, xj3 - xi], axis=2).reshape(-1, 2 * H)
    mu = jnp.mean(f, axis=0)
    var = jnp.mean((f - mu) ** 2, axis=0)
    msg = jnp.maximum((f - mu) * lax.rsqrt(var + EPS) * dg + db, 0.0) @ dW.T
    return jnp.max(msg.reshape(N, K, H), axis=1)


def _static_conv(z, src_s, dst_s, dst2d, end_idx_pad, has, g, b, W, N):
    C = z.shape[1]
    zi = _gather_rows(z, dst_s)
    zj = _gather_rows(z, src_s)
    st = _bn_stats(_stats_static, (zi, zj), float(dst_s.shape[0]), g, b, C)
    out2 = _msg_static(zi, zj, dst2d, st, W.T)
    gathered = _gather_rows(out2, end_idx_pad)[:N]
    return jnp.where(has[:, None], gathered, 0.0)


def kernel(x, edge_index, batch, Wa, ba, Wv, bv, g1, b1, W1, g2, b2, W2,
           g3, b3, W3, g4, b4, W4, dg1, db1, dW1, dg2, db2, dW2, dg3, db3,
           dW3, dg4, db4, dW4, Wfc, bfc):
    N = x.shape[0]
    E = edge_index.shape[1]
    Npad = ((N + 2559) // 2560) * 2560 if N > 2560 else ((N + 127) // 128) * 128

    perm = jnp.argsort(edge_index[1])
    dst_s = edge_index[1][perm]
    src_s = edge_index[0][perm]
    dst2d = dst_s[:, None]
    ar = jnp.arange(N, dtype=jnp.int32)
    ends = jnp.searchsorted(dst_s, ar, side="right").astype(jnp.int32)
    starts = jnp.searchsorted(dst_s, ar, side="left").astype(jnp.int32)
    has = ends > starts
    Ngpad = ((N + 255) // 256) * 256
    end_idx_pad = jnp.pad(jnp.where(has, ends - 1, 0), (0, Ngpad - N))

    brow = batch[:, None]
    bcol8 = jnp.broadcast_to(jnp.pad(batch, (0, Npad - N))[None, :], (8, Npad))

    x0 = _proj(x, Wa.T, _pad8(ba), Wv.T, _pad8(bv))

    x1d = _dyn_conv(x0, brow, bcol8, dg1, db1, dW1, N, Npad)
    x2d = _dyn_conv(x1d, brow, bcol8, dg2, db2, dW2, N, Npad)
    x3d = _dyn_conv(x2d, brow, bcol8, dg3, db3, dW3, N, Npad)
    x4d = _dyn_conv(x3d, brow, bcol8, dg4, db4, dW4, N, Npad)

    sargs = (src_s, dst_s, dst2d, end_idx_pad, has)
    x1 = _static_conv(x0, *sargs, g1, b1, W1, N)
    x2 = _static_conv(jnp.concatenate([x1d, x1], axis=1), *sargs, g2, b2, W2, N)
    x3 = _static_conv(jnp.concatenate([x2d, x2], axis=1), *sargs, g3, b3, W3, N)
    x4 = _static_conv(jnp.concatenate([x3d, x3], axis=1), *sargs, g4, b4, W4, N)

    x4c = jnp.concatenate([x4d, x4], axis=1)
    WfcT8 = jnp.pad(Wfc.T, ((0, 0), (0, 8 - Wfc.shape[0])))
    out = _fc(x4c, WfcT8, jnp.pad(_pad8(bfc), ((0, 0), (0, 8 - bfc.shape[0]))))
    return out[:, :Wfc.shape[0]]
